# baseline (speedup 1.0000x reference)
import functools

import jax
import jax.numpy as jnp
from jax import lax
from jax.experimental import pallas as pl
from jax.experimental.pallas import tpu as pltpu
from jax.experimental.pallas import tpu_sc as plsc

D_MODEL = 128
SCALE = float(D_MODEL) ** 0.5

_info = plsc.get_sparse_core_info()
_NC, _NS = _info.num_cores, _info.num_subcores
_NW = _NC * _NS

_B = 4096 * 200
_CHUNK = 128
_ROWS_PER_W = _B // _NW
_CHUNKS_PER_W = _ROWS_PER_W // _CHUNK


def _scale_body(t_ref, o_ref):
    o_ref[...] = t_ref[...] * SCALE


def _scale_table(table):
    rows = table.shape[0]
    blk = 2000
    return pl.pallas_call(
        _scale_body,
        out_shape=jax.ShapeDtypeStruct(table.shape, table.dtype),
        grid=(rows // blk,),
        in_specs=[pl.BlockSpec((blk, D_MODEL), lambda i: (i, 0))],
        out_specs=pl.BlockSpec((blk, D_MODEL), lambda i: (i, 0)),
    )(table)


@functools.partial(
    pl.kernel,
    mesh=plsc.VectorSubcoreMesh(core_axis_name="c", subcore_axis_name="s"),
    out_type=jax.ShapeDtypeStruct((_B, D_MODEL), jnp.float32),
    scratch_types=(
        [pltpu.VMEM((_CHUNKS_PER_W, _CHUNK), jnp.int32)]
        + [pltpu.VMEM((_CHUNK, D_MODEL), jnp.float32) for _ in range(4)]
        + [pltpu.SemaphoreType.DMA for _ in range(8)]
    ),
)
def _sc_gather(table_hbm, idx_hbm, out_hbm, idx_v, b0, b1, b2, b3,
               g0, g1, g2, g3, s0, s1, s2, s3):
    bufs = [b0, b1, b2, b3]
    gsems = [g0, g1, g2, g3]
    ssems = [s0, s1, s2, s3]
    wid = lax.axis_index("s") * _NC + lax.axis_index("c")
    ibase = wid * _CHUNKS_PER_W
    obase = wid * _ROWS_PER_W
    pltpu.sync_copy(idx_hbm.at[pl.ds(ibase, _CHUNKS_PER_W)], idx_v)

    def gather(j, k):
        pltpu.make_async_copy(table_hbm.at[idx_v.at[j]], bufs[k], gsems[k]).start()

    def gwait(k):
        pltpu.make_async_copy(table_hbm.at[idx_v.at[0]], bufs[k], gsems[k]).wait()

    def scatter(j, k):
        pltpu.make_async_copy(
            bufs[k], out_hbm.at[pl.ds(obase + j * _CHUNK, _CHUNK)], ssems[k]
        ).start()

    def swait(k):
        pltpu.make_async_copy(
            bufs[k], out_hbm.at[pl.ds(obase, _CHUNK)], ssems[k]
        ).wait()

    for k in range(4):
        gather(k, k)
        if k >= 2:
            gwait(k - 2)
            scatter(k - 2, k - 2)

    ngroups = _CHUNKS_PER_W // 4

    def body(i, carry):
        j0 = 4 * i
        for k in range(4):
            j = j0 + k
            swait(k)
            gather(j, k)
            gwait((k + 2) % 4)
            scatter(j - 2, (k + 2) % 4)
        return carry

    lax.fori_loop(1, ngroups, body, 0)

    for k in (2, 3):
        gwait(k)
        scatter(_CHUNKS_PER_W - 4 + k, k)
    for k in range(4):
        swait(k)


def kernel(x, table):
    table_scaled = _scale_table(table)
    idx = x.reshape(_B // _CHUNK, _CHUNK).astype(jnp.int32)
    out = _sc_gather(table_scaled, idx)
    return out.reshape(x.shape[0], x.shape[1], D_MODEL)


# speedup vs baseline: 1.1475x; 1.1475x over previous
import functools

import jax
import jax.numpy as jnp
from jax import lax
from jax.experimental import pallas as pl
from jax.experimental.pallas import tpu as pltpu
from jax.experimental.pallas import tpu_sc as plsc

D_MODEL = 128
SCALE = float(D_MODEL) ** 0.5

_info = plsc.get_sparse_core_info()
_NC, _NS = _info.num_cores, _info.num_subcores
_NW = _NC * _NS

_B = 4096 * 200
_CHUNK = 128
_ROWS_PER_W = _B // _NW
_CHUNKS_PER_W = _ROWS_PER_W // _CHUNK


@functools.partial(
    pl.kernel,
    mesh=plsc.VectorSubcoreMesh(core_axis_name="c", subcore_axis_name="s"),
    out_type=jax.ShapeDtypeStruct((_B, D_MODEL), jnp.float32),
    scratch_types=(
        [pltpu.VMEM((_CHUNKS_PER_W, _CHUNK), jnp.int32)]
        + [pltpu.VMEM((_CHUNK, D_MODEL), jnp.float32) for _ in range(4)]
        + [pltpu.SemaphoreType.DMA for _ in range(8)]
    ),
)
def _sc_gather(table_hbm, idx_hbm, out_hbm, idx_v, b0, b1, b2, b3,
               g0, g1, g2, g3, s0, s1, s2, s3):
    bufs = [b0, b1, b2, b3]
    gsems = [g0, g1, g2, g3]
    ssems = [s0, s1, s2, s3]
    wid = lax.axis_index("s") * _NC + lax.axis_index("c")
    ibase = wid * _CHUNKS_PER_W
    obase = wid * _ROWS_PER_W
    pltpu.sync_copy(idx_hbm.at[pl.ds(ibase, _CHUNKS_PER_W)], idx_v)

    def gather(j, k):
        pltpu.make_async_copy(table_hbm.at[idx_v.at[j]], bufs[k], gsems[k]).start()

    def gwait(k):
        pltpu.make_async_copy(table_hbm.at[idx_v.at[0]], bufs[k], gsems[k]).wait()

    def scatter(j, k):
        pltpu.make_async_copy(
            bufs[k], out_hbm.at[pl.ds(obase + j * _CHUNK, _CHUNK)], ssems[k]
        ).start()

    def swait(k):
        pltpu.make_async_copy(
            bufs[k], out_hbm.at[pl.ds(obase, _CHUNK)], ssems[k]
        ).wait()

    def scale_buf(k):
        buf = bufs[k]

        def row_body(i, carry):
            for r_off in range(2):
                r = 2 * i + r_off
                for c in range(D_MODEL // 16):
                    sl = pl.ds(c * 16, 16)
                    buf[r, sl] = buf[r, sl] * SCALE
            return carry

        lax.fori_loop(0, _CHUNK // 2, row_body, 0)

    for k in range(4):
        gather(k, k)
        if k >= 2:
            gwait(k - 2)
            scale_buf(k - 2)
            scatter(k - 2, k - 2)

    ngroups = _CHUNKS_PER_W // 4

    def body(i, carry):
        j0 = 4 * i
        for k in range(4):
            j = j0 + k
            swait(k)
            gather(j, k)
            gwait((k + 2) % 4)
            scale_buf((k + 2) % 4)
            scatter(j - 2, (k + 2) % 4)
        return carry

    lax.fori_loop(1, ngroups, body, 0)

    for k in (2, 3):
        gwait(k)
        scale_buf(k)
        scatter(_CHUNKS_PER_W - 4 + k, k)
    for k in range(4):
        swait(k)


def kernel(x, table):
    idx = x.reshape(_B // _CHUNK, _CHUNK).astype(jnp.int32)
    out = _sc_gather(table, idx)
    return out.reshape(x.shape[0], x.shape[1], D_MODEL)


# speedup vs baseline: 1.1479x; 1.0003x over previous
import functools

import jax
import jax.numpy as jnp
from jax import lax
from jax.experimental import pallas as pl
from jax.experimental.pallas import tpu as pltpu
from jax.experimental.pallas import tpu_sc as plsc

D_MODEL = 128
SCALE = float(D_MODEL) ** 0.5

_info = plsc.get_sparse_core_info()
_NC, _NS = _info.num_cores, _info.num_subcores
_NW = _NC * _NS

_B = 4096 * 200
_CHUNK = 128
_ROWS_PER_W = _B // _NW
_CHUNKS_PER_W = _ROWS_PER_W // _CHUNK


@functools.partial(
    pl.kernel,
    mesh=plsc.VectorSubcoreMesh(core_axis_name="c", subcore_axis_name="s"),
    out_type=jax.ShapeDtypeStruct((_B, D_MODEL), jnp.float32),
    scratch_types=(
        [pltpu.VMEM((_CHUNKS_PER_W, _CHUNK), jnp.int32)]
        + [pltpu.VMEM((_CHUNK, D_MODEL), jnp.float32) for _ in range(4)]
        + [pltpu.SemaphoreType.DMA for _ in range(8)]
    ),
)
def _sc_gather(table_hbm, idx_hbm, out_hbm, idx_v, b0, b1, b2, b3,
               g0, g1, g2, g3, s0, s1, s2, s3):
    bufs = [b0, b1, b2, b3]
    gsems = [g0, g1, g2, g3]
    ssems = [s0, s1, s2, s3]
    wid = lax.axis_index("s") * _NC + lax.axis_index("c")
    ibase = wid * _CHUNKS_PER_W
    obase = wid * _ROWS_PER_W
    pltpu.sync_copy(idx_hbm.at[pl.ds(ibase, _CHUNKS_PER_W)], idx_v)

    def gather(j, k):
        pltpu.make_async_copy(table_hbm.at[idx_v.at[j]], bufs[k], gsems[k]).start()

    def gwait(k):
        pltpu.make_async_copy(table_hbm.at[idx_v.at[0]], bufs[k], gsems[k]).wait()

    def scatter(j, k):
        pltpu.make_async_copy(
            bufs[k], out_hbm.at[pl.ds(obase + j * _CHUNK, _CHUNK)], ssems[k]
        ).start()

    def swait(k):
        pltpu.make_async_copy(
            bufs[k], out_hbm.at[pl.ds(obase, _CHUNK)], ssems[k]
        ).wait()

    def scale_buf(k):
        buf = bufs[k]

        def row_body(i, carry):
            for r_off in range(4):
                r = 4 * i + r_off
                for c in range(D_MODEL // 16):
                    sl = pl.ds(c * 16, 16)
                    buf[r, sl] = buf[r, sl] * SCALE
            return carry

        lax.fori_loop(0, _CHUNK // 4, row_body, 0)

    for k in range(4):
        gather(k, k)
        if k >= 2:
            gwait(k - 2)
            scale_buf(k - 2)
            scatter(k - 2, k - 2)

    ngroups = _CHUNKS_PER_W // 4

    def body(i, carry):
        j0 = 4 * i
        for k in range(4):
            j = j0 + k
            swait(k)
            gather(j, k)
            gwait((k + 2) % 4)
            scale_buf((k + 2) % 4)
            scatter(j - 2, (k + 2) % 4)
        return carry

    lax.fori_loop(1, ngroups, body, 0)

    for k in (2, 3):
        gwait(k)
        scale_buf(k)
        scatter(_CHUNKS_PER_W - 4 + k, k)
    for k in range(4):
        swait(k)


def kernel(x, table):
    idx = x.reshape(_B // _CHUNK, _CHUNK).astype(jnp.int32)
    out = _sc_gather(table, idx)
    return out.reshape(x.shape[0], x.shape[1], D_MODEL)
